# Initial kernel scaffold; baseline (speedup 1.0000x reference)
import jax
import jax.numpy as jnp
from jax.experimental import pallas as pl


def kernel(node_ft, batch_index, num_graphs, W):
    raise NotImplementedError("write your pallas kernel here")


# speedup vs baseline: 4.3964x; 4.3964x over previous
import numpy as np
import jax
import jax.numpy as jnp
from jax.experimental import pallas as pl
from jax.experimental.pallas import tpu as pltpu

D = 128
P = D * (D + 1) // 2
NK = D // 2 + 1
G = 512
B = 400

_SELU_SCALE = 1.0507009873554804934193349852946
_SELU_ALPHA = 1.6732632423543772848170429916717
_RSQRT_P = 1.0 / np.sqrt(np.float32(P))
_QROOT2 = np.float32(2.0) ** 0.25

_iu, _ju = np.triu_indices(D)
_pair = np.zeros((D, D), np.int32)
_pair[_iu, _ju] = np.arange(P, dtype=np.int32)
_pair[_ju, _iu] = np.arange(P, dtype=np.int32)
_ii = np.tile(np.arange(D)[None, :], (NK, 1))
_jj = (_ii + np.arange(NK)[:, None]) % D
_PIDX = _pair[_ii, _jj]
_MASKK = np.ones((NK, D), np.float32)
_MASKK[NK - 1, D // 2:] = 0.0


def _body(nsteps, f_ref, idx_ref, wk_ref, out_ref, acc_ref, z_ref):
    i = pl.program_id(0)
    F = f_ref[...]
    Fs = F * _QROOT2

    acc2d = (F * F) * wk_ref[0:1, :]
    for k in range(1, NK):
        Fr = jnp.concatenate([Fs[:, k:], Fs[:, :k]], axis=1)
        T = Fs * Fr
        E = jnp.exp(T) * _SELU_ALPHA - _SELU_ALPHA
        S = jnp.where(T > 0.0, T, E)
        acc2d = acc2d + S * wk_ref[k:k + 1, :]
    logit = jnp.sum(acc2d, axis=1, keepdims=True)
    e = jnp.exp(logit)

    idx = idx_ref[...].reshape(1, B)
    onehot_t = (jax.lax.broadcasted_iota(jnp.int32, (G, B), 0) == idx
                ).astype(jnp.float32)

    @pl.when(i == 0)
    def _init():
        acc_ref[...] = jnp.zeros_like(acc_ref)
        z_ref[...] = jnp.zeros_like(z_ref)

    acc_ref[...] += jnp.dot(onehot_t, F * e, preferred_element_type=jnp.float32)
    z_ref[...] += jnp.dot(onehot_t, jnp.broadcast_to(e, (B, D)),
                          preferred_element_type=jnp.float32)

    @pl.when(i == nsteps - 1)
    def _finish():
        out_ref[...] = acc_ref[...] / jnp.maximum(z_ref[...], 1e-30)


def kernel(node_ft, batch_index, num_graphs, W):
    n = node_ft.shape[0]
    nsteps = n // B
    wk = W[_PIDX] * jnp.asarray(_MASKK) * (_SELU_SCALE * _RSQRT_P)
    wk = jnp.pad(wk, ((0, 72 - NK), (0, 0)))
    idx3 = batch_index.astype(jnp.int32).reshape(nsteps, 1, B)

    out = pl.pallas_call(
        lambda *refs: _body(nsteps, *refs),
        grid=(nsteps,),
        in_specs=[
            pl.BlockSpec((B, D), lambda i: (i, 0)),
            pl.BlockSpec((1, 1, B), lambda i: (i, 0, 0)),
            pl.BlockSpec((72, D), lambda i: (0, 0)),
        ],
        out_specs=pl.BlockSpec((G, D), lambda i: (0, 0)),
        out_shape=jax.ShapeDtypeStruct((G, D), jnp.float32),
        scratch_shapes=[
            pltpu.VMEM((G, D), jnp.float32),
            pltpu.VMEM((G, D), jnp.float32),
        ],
    )(node_ft, idx3, wk)

    valid = jnp.arange(G) < num_graphs
    return jnp.where(valid[:, None], out, jnp.zeros_like(out))
